# baseline (speedup 1.0000x reference)
import jax
import jax.numpy as jnp
from jax.experimental import pallas as pl

DEPTH = 1000
ROWS_PER_BLOCK = 1024


def _onehot_block(idx_ref, out_ref):
    idx = idx_ref[...]
    iota = jax.lax.broadcasted_iota(idx.dtype, out_ref.shape, 1)
    out_ref[...] = (idx == iota).astype(jnp.float32)


def kernel(inputs):
    n, m = inputs.shape
    rows = n * m
    b = ROWS_PER_BLOCK
    flat_idx = inputs.reshape(rows, 1)
    out = pl.pallas_call(
        _onehot_block,
        grid=(rows // b,),
        in_specs=[pl.BlockSpec((b, 1), lambda i: (i, 0))],
        out_specs=pl.BlockSpec((b, DEPTH), lambda i: (i, 0)),
        out_shape=jax.ShapeDtypeStruct((rows, DEPTH), jnp.float32),
    )(flat_idx)
    return out.reshape(n, m, DEPTH)


# speedup vs baseline: 1.0443x; 1.0443x over previous
import jax
import jax.numpy as jnp
from jax.experimental import pallas as pl
from jax.experimental.pallas import tpu as pltpu

DEPTH = 1000
B = 1024
NBUF = 8
CHUNKS = 80


def _body(idx_ref, out_ref, buf_ref, sems):
    def make_copy(chunk, slot):
        return pltpu.make_async_copy(
            buf_ref.at[slot],
            out_ref.at[pl.ds(chunk * B, B), :],
            sems.at[slot],
        )

    iota = jax.lax.broadcasted_iota(jnp.int32, (B, DEPTH), 1)

    def group(g, carry):
        for slot in range(NBUF):
            chunk = g * NBUF + slot

            @pl.when(g > 0)
            def _wait_prev():
                make_copy(chunk - NBUF, slot).wait()

            row = idx_ref[pl.ds(chunk, 1), :]
            col = jax.lax.transpose(row, (1, 0))
            buf_ref[slot] = (col == iota).astype(jnp.float32)
            make_copy(chunk, slot).start()
        return carry

    jax.lax.fori_loop(0, CHUNKS // NBUF, group, 0)
    for slot in range(NBUF):
        make_copy(CHUNKS - NBUF + slot, slot).wait()


def kernel(inputs):
    n, m = inputs.shape
    rows = n * m
    idx = inputs.reshape(CHUNKS, B)
    out = pl.pallas_call(
        _body,
        in_specs=[pl.BlockSpec(memory_space=pltpu.MemorySpace.VMEM)],
        out_specs=pl.BlockSpec(memory_space=pl.ANY),
        out_shape=jax.ShapeDtypeStruct((rows, DEPTH), jnp.float32),
        scratch_shapes=[
            pltpu.VMEM((NBUF, B, DEPTH), jnp.float32),
            pltpu.SemaphoreType.DMA((NBUF,)),
        ],
        compiler_params=pltpu.CompilerParams(
            vmem_limit_bytes=100 * 1024 * 1024,
        ),
    )(idx)
    return out.reshape(n, m, DEPTH)


# speedup vs baseline: 7.4161x; 7.1016x over previous
import jax
import jax.numpy as jnp
from jax.experimental import pallas as pl

DEPTH = 1000
R = 1024


def _onehot_block(idx_ref, out_ref):
    idx = idx_ref[...]
    iota = jax.lax.broadcasted_iota(idx.dtype, out_ref.shape, 1)
    out_ref[...] = (idx == iota).astype(jnp.float32)


def kernel(inputs):
    n, m = inputs.shape
    idx_t = inputs.T.reshape(m, 1, n)
    out_t = pl.pallas_call(
        _onehot_block,
        grid=(m, n // R),
        in_specs=[pl.BlockSpec((1, 1, R), lambda j, i: (j, 0, i))],
        out_specs=pl.BlockSpec((1, DEPTH, R), lambda j, i: (j, 0, i)),
        out_shape=jax.ShapeDtypeStruct((m, DEPTH, n), jnp.float32),
    )(idx_t)
    return out_t.transpose(2, 0, 1)
